# Initial kernel scaffold; baseline (speedup 1.0000x reference)
import jax
import jax.numpy as jnp
from jax.experimental import pallas as pl


def kernel(x, W, b):
    raise NotImplementedError("write your pallas kernel here")


# speedup vs baseline: 1.0001x; 1.0001x over previous
import functools

import jax
import jax.numpy as jnp
from jax import lax
from jax.experimental import pallas as pl
from jax.experimental.pallas import tpu as pltpu

SP_RATIO_K = 4


def _proj_body(x_ref, w_ref, b_ref, val_ref, idx_ref):
    x = x_ref[0]
    w = w_ref[...]
    val = lax.dot_general(
        w, x, (((1,), (0,)), ((), ())),
        preferred_element_type=jnp.float32,
    ) + b_ref[...]
    val_ref[0] = val
    r = w.shape[0]
    riota = lax.broadcasted_iota(jnp.int32, val.shape, 0)
    m = jnp.max(val, axis=0, keepdims=True)
    idx = jnp.min(jnp.where(val == m, riota, r), axis=0)
    idx_ref[0, 0] = idx[None, :]


def _project(x, W, b):
    B, D, N = x.shape
    R = W.shape[0]
    NB = 2048
    grid = (B, N // NB)
    val, idx = pl.pallas_call(
        _proj_body,
        grid=grid,
        in_specs=[
            pl.BlockSpec((1, D, NB), lambda i, j: (i, 0, j)),
            pl.BlockSpec((R, D), lambda i, j: (0, 0)),
            pl.BlockSpec((R, 1), lambda i, j: (0, 0)),
        ],
        out_specs=[
            pl.BlockSpec((1, R, NB), lambda i, j: (i, 0, j)),
            pl.BlockSpec((1, 1, 1, NB), lambda i, j: (i, j, 0, 0)),
        ],
        out_shape=[
            jax.ShapeDtypeStruct((B, R, N), jnp.float32),
            jax.ShapeDtypeStruct((B, N // NB, 1, NB), jnp.int32),
        ],
    )(x, W, b.reshape(R, 1))
    return val, idx.reshape(B, N)


def kernel(x, W, b):
    B, D, N = x.shape
    R = W.shape[0]
    P = N // SP_RATIO_K
    val_sort, idx_sort = _project(x, W, b)
    _, idx_filter = lax.top_k(val_sort, P)
    gath = jnp.broadcast_to(idx_filter.reshape(B, 1, R * P), (B, D, R * P))
    FEAT_star = jnp.take_along_axis(x, gath, axis=2).reshape(B, D, R, P)
    idx_star = jnp.broadcast_to(
        idx_filter[:, None, :, :].astype(jnp.float32), (B, R, R, P))
    return (FEAT_star, idx_star, idx_sort)


# speedup vs baseline: 596.5676x; 596.5159x over previous
import functools

import jax
import jax.numpy as jnp
from jax import lax
from jax.experimental import pallas as pl
from jax.experimental.pallas import tpu as pltpu
from jax.experimental.pallas import tpu_sc as plsc

SP_RATIO_K = 4


def _proj_body(x_ref, w_ref, b_ref, val_ref, idx_ref):
    x = x_ref[0]
    w = w_ref[...]
    val = lax.dot_general(
        w, x, (((1,), (0,)), ((), ())),
        preferred_element_type=jnp.float32,
    ) + b_ref[...]
    val_ref[0] = val
    r = w.shape[0]
    riota = lax.broadcasted_iota(jnp.int32, val.shape, 0)
    m = jnp.max(val, axis=0, keepdims=True)
    idx = jnp.min(jnp.where(val == m, riota, r), axis=0)
    idx_ref[0, 0] = idx[None, :]


def _project(x, W, b):
    B, D, N = x.shape
    R = W.shape[0]
    NB = 2048
    grid = (B, N // NB)
    val, idx = pl.pallas_call(
        _proj_body,
        grid=grid,
        in_specs=[
            pl.BlockSpec((1, D, NB), lambda i, j: (i, 0, j)),
            pl.BlockSpec((R, D), lambda i, j: (0, 0)),
            pl.BlockSpec((R, 1), lambda i, j: (0, 0)),
        ],
        out_specs=[
            pl.BlockSpec((1, R, NB), lambda i, j: (i, 0, j)),
            pl.BlockSpec((1, 1, 1, NB), lambda i, j: (i, j, 0, 0)),
        ],
        out_shape=[
            jax.ShapeDtypeStruct((B, R, N), jnp.float32),
            jax.ShapeDtypeStruct((B, N // NB, 1, NB), jnp.int32),
        ],
    )(x, W, b.reshape(R, 1))
    return val, idx.reshape(B, N)


def _make_gather(B, D, N, RP):
    info = plsc.get_sparse_core_info()
    NC, NS, L = info.num_cores, info.num_subcores, info.num_lanes
    NW = NC * NS
    assert D % (NW // B) == 0
    d_per_w = D // (NW // B)
    mesh = plsc.VectorSubcoreMesh(core_axis_name="c", subcore_axis_name="s")

    @functools.partial(
        pl.kernel, mesh=mesh,
        out_type=jax.ShapeDtypeStruct((B, D, RP), jnp.float32),
        compiler_params=pltpu.CompilerParams(needs_layout_passes=False),
        scratch_types=[
            pltpu.VMEM((RP,), jnp.int32),
            pltpu.VMEM((N,), jnp.float32),
            pltpu.VMEM((RP,), jnp.float32),
        ],
    )
    def gather(x_hbm, idx_hbm, out_hbm, idx_v, row_v, out_v):
        wid = lax.axis_index("s") * NC + lax.axis_index("c")
        b = wid // (NW // B)
        d0 = (wid % (NW // B)) * d_per_w
        pltpu.sync_copy(idx_hbm.at[b], idx_v)

        def per_d(di, carry):
            d = d0 + di
            pltpu.sync_copy(x_hbm.at[b, d], row_v)

            def per_j(j, c):
                idxv = idx_v[pl.ds(j * L, L)]
                out_v[pl.ds(j * L, L)] = plsc.load_gather(row_v, [idxv])
                return c

            lax.fori_loop(0, RP // L, per_j, 0, unroll=8)
            pltpu.sync_copy(out_v, out_hbm.at[b, d])
            return carry

        lax.fori_loop(0, d_per_w, per_d, 0)

    return gather


def kernel(x, W, b):
    B, D, N = x.shape
    R = W.shape[0]
    P = N // SP_RATIO_K
    val_sort, idx_sort = _project(x, W, b)
    _, idx_filter = lax.top_k(val_sort, P)
    feat = _make_gather(B, D, N, R * P)(x, idx_filter.reshape(B, R * P))
    FEAT_star = feat.reshape(B, D, R, P)
    idx_star = jnp.broadcast_to(
        idx_filter[:, None, :, :].astype(jnp.float32), (B, R, R, P))
    return (FEAT_star, idx_star, idx_sort)


# speedup vs baseline: 601.1774x; 1.0077x over previous
import functools

import jax
import jax.numpy as jnp
from jax import lax
from jax.experimental import pallas as pl
from jax.experimental.pallas import tpu as pltpu
from jax.experimental.pallas import tpu_sc as plsc

SP_RATIO_K = 4


def _proj_body(x_ref, w_ref, b_ref, val_ref, idx_ref):
    x = x_ref[0]
    w = w_ref[...]
    val = lax.dot_general(
        w, x, (((1,), (0,)), ((), ())),
        preferred_element_type=jnp.float32,
    ) + b_ref[...]
    val_ref[0] = val
    r = w.shape[0]
    riota = lax.broadcasted_iota(jnp.int32, val.shape, 0)
    m = jnp.max(val, axis=0, keepdims=True)
    idx = jnp.min(jnp.where(val == m, riota, r), axis=0)
    idx_ref[0, 0] = idx[None, :]


def _project(x, W, b):
    B, D, N = x.shape
    R = W.shape[0]
    NB = 2048
    grid = (B, N // NB)
    val, idx = pl.pallas_call(
        _proj_body,
        grid=grid,
        in_specs=[
            pl.BlockSpec((1, D, NB), lambda i, j: (i, 0, j)),
            pl.BlockSpec((R, D), lambda i, j: (0, 0)),
            pl.BlockSpec((R, 1), lambda i, j: (0, 0)),
        ],
        out_specs=[
            pl.BlockSpec((1, R, NB), lambda i, j: (i, 0, j)),
            pl.BlockSpec((1, 1, 1, NB), lambda i, j: (i, j, 0, 0)),
        ],
        out_shape=[
            jax.ShapeDtypeStruct((B, R, N), jnp.float32),
            jax.ShapeDtypeStruct((B, N // NB, 1, NB), jnp.int32),
        ],
    )(x, W, b.reshape(R, 1))
    return val, idx.reshape(B, N)


def _make_sort(BR, N, P):
    info = plsc.get_sparse_core_info()
    NC, NS, L = info.num_cores, info.num_subcores, info.num_lanes
    NW = NC * NS
    RPW = BR // NW
    SEG = N // L
    NV = N // L
    NBINS = 256
    mesh = plsc.VectorSubcoreMesh(core_axis_name="c", subcore_axis_name="s")

    @functools.partial(
        pl.kernel, mesh=mesh,
        out_type=jax.ShapeDtypeStruct((BR, P), jnp.int32),
        compiler_params=pltpu.CompilerParams(needs_layout_passes=False),
        scratch_types=[
            pltpu.VMEM((N,), jnp.float32),
            pltpu.VMEM((N,), jnp.int32),
            pltpu.VMEM((N,), jnp.int32),
            pltpu.VMEM((N,), jnp.int32),
            pltpu.VMEM((N,), jnp.int32),
            pltpu.VMEM((NBINS * L,), jnp.int32),
            pltpu.VMEM((NBINS * L,), jnp.int32),
            pltpu.VMEM((N,), jnp.int32),
        ],
    )
    def sortk(val_hbm, out_hbm, row_v, key_a, idx_a, key_b, idx_b, hist, off,
              pos_buf):
        wid = lax.axis_index("s") * NC + lax.axis_index("c")
        lanes = lax.iota(jnp.int32, L)
        ones = jnp.ones((L,), jnp.int32)
        zeros = jnp.zeros((L,), jnp.int32)

        def per_row(rr, carry0):
            row = wid * RPW + rr
            pltpu.sync_copy(val_hbm.at[row], row_v)

            def build(v, c):
                f = row_v[pl.ds(v * L, L)]
                u = plsc.bitcast(f, jnp.int32)
                key = jnp.where(u < 0, ~u, u ^ jnp.int32(-2147483648))
                key_a[pl.ds(v * L, L)] = key
                idx_a[pl.ds(v * L, L)] = lanes + v * L
                return c

            lax.fori_loop(0, NV, build, 0, unroll=8)

            for p in range(4):
                src_k, src_i = (key_a, idx_a) if p % 2 == 0 else (key_b, idx_b)
                dst_k, dst_i = (key_b, idx_b) if p % 2 == 0 else (key_a, idx_a)
                shift = jnp.int32(8 * p)

                def zero(h, c):
                    hist[pl.ds(h * L, L)] = zeros
                    return c

                lax.fori_loop(0, NBINS, zero, 0, unroll=8)

                def histo(v, c):
                    kv = plsc.load_gather(src_k, [lanes * SEG + v])
                    d = lax.shift_right_logical(kv, shift) & 0xFF
                    plsc.addupdate_scatter(hist, [d * L + lanes], ones)
                    return c

                lax.fori_loop(0, NV, histo, 0, unroll=8)

                def scan_b(h, carry):
                    d = jnp.int32(NBINS - 1) - h
                    hv = plsc.load_gather(hist, [d * L + lanes])
                    incl = plsc.cumsum(hv)
                    excl = incl - hv + carry
                    plsc.store_scatter(off, [d * L + lanes], excl)
                    return carry + jnp.sum(hv)

                lax.fori_loop(0, NBINS, scan_b, jnp.int32(0), unroll=2)

                def perm_pos(v, c):
                    g = lanes * SEG + v
                    kv = plsc.load_gather(src_k, [g])
                    d = lax.shift_right_logical(kv, shift) & 0xFF
                    slot = d * L + lanes
                    pos = plsc.load_gather(off, [slot]) & jnp.int32(N - 1)
                    plsc.store_scatter(pos_buf, [g], pos)
                    plsc.addupdate_scatter(off, [slot], ones)
                    return c

                lax.fori_loop(0, NV, perm_pos, 0)

                def perm_move(v, c):
                    s = pl.ds(v * L, L)
                    pos = pos_buf[s]
                    kv = src_k[s]
                    iv = src_i[s]
                    plsc.store_scatter(dst_k, [pos], kv)
                    plsc.store_scatter(dst_i, [pos], iv)
                    return c

                lax.fori_loop(0, NV, perm_move, 0)

            pltpu.sync_copy(idx_a.at[pl.ds(0, P)], out_hbm.at[row])
            return carry0

        lax.fori_loop(0, RPW, per_row, 0)

    return sortk


def _make_gather(B, D, N, RP):
    info = plsc.get_sparse_core_info()
    NC, NS, L = info.num_cores, info.num_subcores, info.num_lanes
    NW = NC * NS
    d_per_w = D // (NW // B)
    mesh = plsc.VectorSubcoreMesh(core_axis_name="c", subcore_axis_name="s")

    @functools.partial(
        pl.kernel, mesh=mesh,
        out_type=jax.ShapeDtypeStruct((B, D, RP), jnp.float32),
        compiler_params=pltpu.CompilerParams(needs_layout_passes=False),
        scratch_types=[
            pltpu.VMEM((RP,), jnp.int32),
            pltpu.VMEM((N,), jnp.float32),
            pltpu.VMEM((RP,), jnp.float32),
        ],
    )
    def gather(x_hbm, idx_hbm, out_hbm, idx_v, row_v, out_v):
        wid = lax.axis_index("s") * NC + lax.axis_index("c")
        b = wid // (NW // B)
        d0 = (wid % (NW // B)) * d_per_w
        pltpu.sync_copy(idx_hbm.at[b], idx_v)

        def per_d(di, carry):
            d = d0 + di
            pltpu.sync_copy(x_hbm.at[b, d], row_v)

            def per_j(j, c):
                idxv = idx_v[pl.ds(j * L, L)]
                out_v[pl.ds(j * L, L)] = plsc.load_gather(row_v, [idxv])
                return c

            lax.fori_loop(0, RP // L, per_j, 0, unroll=8)

            pltpu.sync_copy(out_v, out_hbm.at[b, d])
            return carry

        lax.fori_loop(0, d_per_w, per_d, 0)

    return gather


def kernel(x, W, b):
    B, D, N = x.shape
    R = W.shape[0]
    P = N // SP_RATIO_K
    val_sort, idx_sort = _project(x, W, b)
    idx_filter = _make_sort(B * R, N, P)(val_sort.reshape(B * R, N))
    feat = _make_gather(B, D, N, R * P)(x, idx_filter.reshape(B, R * P))
    FEAT_star = feat.reshape(B, D, R, P)
    idx_star = jnp.broadcast_to(
        idx_filter.reshape(B, R, P)[:, None, :, :].astype(jnp.float32),
        (B, R, R, P))
    return (FEAT_star, idx_star, idx_sort)


# speedup vs baseline: 715.2124x; 1.1897x over previous
import functools

import jax
import jax.numpy as jnp
from jax import lax
from jax.experimental import pallas as pl
from jax.experimental.pallas import tpu as pltpu
from jax.experimental.pallas import tpu_sc as plsc

SP_RATIO_K = 4


def _proj_body(x_ref, w_ref, b_ref, val_ref, idx_ref):
    x = x_ref[0]
    w = w_ref[...]
    val = lax.dot_general(
        w, x, (((1,), (0,)), ((), ())),
        preferred_element_type=jnp.float32,
    ) + b_ref[...]
    val_ref[0] = val
    r = w.shape[0]
    riota = lax.broadcasted_iota(jnp.int32, val.shape, 0)
    m = jnp.max(val, axis=0, keepdims=True)
    idx = jnp.min(jnp.where(val == m, riota, r), axis=0)
    idx_ref[0, 0] = idx[None, :]


def _project(x, W, b):
    B, D, N = x.shape
    R = W.shape[0]
    NB = 2048
    grid = (B, N // NB)
    val, idx = pl.pallas_call(
        _proj_body,
        grid=grid,
        in_specs=[
            pl.BlockSpec((1, D, NB), lambda i, j: (i, 0, j)),
            pl.BlockSpec((R, D), lambda i, j: (0, 0)),
            pl.BlockSpec((R, 1), lambda i, j: (0, 0)),
        ],
        out_specs=[
            pl.BlockSpec((1, R, NB), lambda i, j: (i, 0, j)),
            pl.BlockSpec((1, 1, 1, NB), lambda i, j: (i, j, 0, 0)),
        ],
        out_shape=[
            jax.ShapeDtypeStruct((B, R, N), jnp.float32),
            jax.ShapeDtypeStruct((B, N // NB, 1, NB), jnp.int32),
        ],
    )(x, W, b.reshape(R, 1))
    return val, idx.reshape(B, N)


def _make_sort(BR, N, P):
    info = plsc.get_sparse_core_info()
    NC, NS, L = info.num_cores, info.num_subcores, info.num_lanes
    NW = NC * NS
    RPW = BR // NW
    SEG = N // L
    NV = N // L
    NBINS = 256
    mesh = plsc.VectorSubcoreMesh(core_axis_name="c", subcore_axis_name="s")

    @functools.partial(
        pl.kernel, mesh=mesh,
        out_type=jax.ShapeDtypeStruct((BR, P), jnp.int32),
        compiler_params=pltpu.CompilerParams(needs_layout_passes=False),
        scratch_types=[
            pltpu.VMEM((N,), jnp.float32),
            pltpu.VMEM((N,), jnp.int32),
            pltpu.VMEM((N,), jnp.int32),
            pltpu.VMEM((N,), jnp.int32),
            pltpu.VMEM((N,), jnp.int32),
            pltpu.VMEM((NBINS * L,), jnp.int32),
            pltpu.VMEM((NBINS * L,), jnp.int32),
            pltpu.VMEM((N,), jnp.int32),
        ],
    )
    def sortk(val_hbm, out_hbm, row_v, key_a, idx_a, key_b, idx_b, hist, off,
              pos_buf):
        wid = lax.axis_index("s") * NC + lax.axis_index("c")
        lanes = lax.iota(jnp.int32, L)
        ones = jnp.ones((L,), jnp.int32)
        zeros = jnp.zeros((L,), jnp.int32)

        def per_row(rr, carry0):
            row = wid * RPW + rr
            pltpu.sync_copy(val_hbm.at[row], row_v)

            def build(v, c):
                f = row_v[pl.ds(v * L, L)]
                u = plsc.bitcast(f, jnp.int32)
                key = jnp.where(u < 0, ~u, u ^ jnp.int32(-2147483648))
                key_a[pl.ds(v * L, L)] = key
                idx_a[pl.ds(v * L, L)] = lanes + v * L
                return c

            lax.fori_loop(0, NV, build, 0, unroll=8)

            for p in range(4):
                src_k, src_i = (key_a, idx_a) if p % 2 == 0 else (key_b, idx_b)
                dst_k, dst_i = (key_b, idx_b) if p % 2 == 0 else (key_a, idx_a)
                shift = jnp.int32(8 * p)

                def zero(h, c):
                    hist[pl.ds(h * L, L)] = zeros
                    return c

                lax.fori_loop(0, NBINS, zero, 0, unroll=8)

                def histo(v, c):
                    kv = plsc.load_gather(src_k, [lanes * SEG + v])
                    d = lax.shift_right_logical(kv, shift) & 0xFF
                    plsc.addupdate_scatter(hist, [d * L + lanes], ones)
                    return c

                lax.fori_loop(0, NV, histo, 0, unroll=8)

                def scan_b(h, carry):
                    d = jnp.int32(NBINS - 1) - h
                    hv = plsc.load_gather(hist, [d * L + lanes])
                    incl = plsc.cumsum(hv)
                    excl = incl - hv + carry
                    plsc.store_scatter(off, [d * L + lanes], excl)
                    return carry + jnp.sum(hv)

                lax.fori_loop(0, NBINS, scan_b, jnp.int32(0), unroll=4)

                def perm_pos(v, c):
                    g = lanes * SEG + v
                    kv = plsc.load_gather(src_k, [g])
                    d = lax.shift_right_logical(kv, shift) & 0xFF
                    slot = d * L + lanes
                    pos = plsc.load_gather(off, [slot]) & jnp.int32(N - 1)
                    plsc.store_scatter(pos_buf, [g], pos)
                    plsc.addupdate_scatter(off, [slot], ones)
                    return c

                lax.fori_loop(0, NV, perm_pos, 0, unroll=4)

                def perm_move(v, c):
                    s = pl.ds(v * L, L)
                    pos = pos_buf[s]
                    kv = src_k[s]
                    iv = src_i[s]
                    plsc.store_scatter(dst_k, [pos], kv)
                    plsc.store_scatter(dst_i, [pos], iv)
                    return c

                lax.fori_loop(0, NV, perm_move, 0, unroll=4)

            pltpu.sync_copy(idx_a.at[pl.ds(0, P)], out_hbm.at[row])
            return carry0

        lax.fori_loop(0, RPW, per_row, 0)

    return sortk


def _make_gather(B, D, N, RP):
    info = plsc.get_sparse_core_info()
    NC, NS, L = info.num_cores, info.num_subcores, info.num_lanes
    NW = NC * NS
    d_per_w = D // (NW // B)
    mesh = plsc.VectorSubcoreMesh(core_axis_name="c", subcore_axis_name="s")

    @functools.partial(
        pl.kernel, mesh=mesh,
        out_type=jax.ShapeDtypeStruct((B, D, RP), jnp.float32),
        compiler_params=pltpu.CompilerParams(needs_layout_passes=False),
        scratch_types=[
            pltpu.VMEM((RP,), jnp.int32),
            pltpu.VMEM((N,), jnp.float32),
            pltpu.VMEM((N,), jnp.float32),
            pltpu.VMEM((RP,), jnp.float32),
            pltpu.VMEM((RP,), jnp.float32),
        ],
    )
    def gather(x_hbm, idx_hbm, out_hbm, idx_v, row0_v, row1_v, out0_v, out1_v):
        wid = lax.axis_index("s") * NC + lax.axis_index("c")
        b = wid // (NW // B)
        d0 = (wid % (NW // B)) * d_per_w
        pltpu.sync_copy(idx_hbm.at[b], idx_v)

        def per_d(di, carry):
            d = d0 + di * 2
            pltpu.sync_copy(x_hbm.at[b, d], row0_v)
            pltpu.sync_copy(x_hbm.at[b, d + 1], row1_v)

            def per_j(j, c):
                s = pl.ds(j * L, L)
                idxv = idx_v[s]
                out0_v[s] = plsc.load_gather(row0_v, [idxv])
                out1_v[s] = plsc.load_gather(row1_v, [idxv])
                return c

            lax.fori_loop(0, RP // L, per_j, 0, unroll=8)

            pltpu.sync_copy(out0_v, out_hbm.at[b, d])
            pltpu.sync_copy(out1_v, out_hbm.at[b, d + 1])
            return carry

        lax.fori_loop(0, d_per_w // 2, per_d, 0)

    return gather


def kernel(x, W, b):
    B, D, N = x.shape
    R = W.shape[0]
    P = N // SP_RATIO_K
    val_sort, idx_sort = _project(x, W, b)
    idx_filter = _make_sort(B * R, N, P)(val_sort.reshape(B * R, N))
    feat = _make_gather(B, D, N, R * P)(x, idx_filter.reshape(B, R * P))
    FEAT_star = feat.reshape(B, D, R, P)
    idx_star = jnp.broadcast_to(
        idx_filter.reshape(B, R, P)[:, None, :, :].astype(jnp.float32),
        (B, R, R, P))
    return (FEAT_star, idx_star, idx_sort)


# speedup vs baseline: 716.8784x; 1.0023x over previous
import functools

import jax
import jax.numpy as jnp
from jax import lax
from jax.experimental import pallas as pl
from jax.experimental.pallas import tpu as pltpu
from jax.experimental.pallas import tpu_sc as plsc

SP_RATIO_K = 4


def _proj_body(x_ref, w_ref, b_ref, val_ref, idx_ref):
    x = x_ref[0]
    w = w_ref[...]
    val = lax.dot_general(
        w, x, (((1,), (0,)), ((), ())),
        preferred_element_type=jnp.float32,
    ) + b_ref[...]
    val_ref[0] = val
    r = w.shape[0]
    riota = lax.broadcasted_iota(jnp.int32, val.shape, 0)
    m = jnp.max(val, axis=0, keepdims=True)
    idx = jnp.min(jnp.where(val == m, riota, r), axis=0)
    idx_ref[0, 0] = idx[None, :]


def _project(x, W, b):
    B, D, N = x.shape
    R = W.shape[0]
    NB = 2048
    grid = (B, N // NB)
    val, idx = pl.pallas_call(
        _proj_body,
        grid=grid,
        in_specs=[
            pl.BlockSpec((1, D, NB), lambda i, j: (i, 0, j)),
            pl.BlockSpec((R, D), lambda i, j: (0, 0)),
            pl.BlockSpec((R, 1), lambda i, j: (0, 0)),
        ],
        out_specs=[
            pl.BlockSpec((1, R, NB), lambda i, j: (i, 0, j)),
            pl.BlockSpec((1, 1, 1, NB), lambda i, j: (i, j, 0, 0)),
        ],
        out_shape=[
            jax.ShapeDtypeStruct((B, R, N), jnp.float32),
            jax.ShapeDtypeStruct((B, N // NB, 1, NB), jnp.int32),
        ],
    )(x, W, b.reshape(R, 1))
    return val, idx.reshape(B, N)


def _make_sort(BR, N, P):
    info = plsc.get_sparse_core_info()
    NC, NS, L = info.num_cores, info.num_subcores, info.num_lanes
    NW = NC * NS
    RPW = BR // NW
    SEG = N // L
    NV = N // L
    NBINS = 256
    mesh = plsc.VectorSubcoreMesh(core_axis_name="c", subcore_axis_name="s")

    @functools.partial(
        pl.kernel, mesh=mesh,
        out_type=jax.ShapeDtypeStruct((BR, P), jnp.int32),
        compiler_params=pltpu.CompilerParams(needs_layout_passes=False),
        scratch_types=[
            pltpu.VMEM((N,), jnp.float32),
            pltpu.VMEM((N,), jnp.int32),
            pltpu.VMEM((N,), jnp.int32),
            pltpu.VMEM((N,), jnp.int32),
            pltpu.VMEM((N,), jnp.int32),
            pltpu.VMEM((NBINS * L,), jnp.int32),
            pltpu.VMEM((NBINS * L,), jnp.int32),
            pltpu.VMEM((N,), jnp.int32),
        ],
    )
    def sortk(val_hbm, out_hbm, row_v, key_a, idx_a, key_b, idx_b, hist, off,
              pos_buf):
        wid = lax.axis_index("s") * NC + lax.axis_index("c")
        lanes = lax.iota(jnp.int32, L)
        ones = jnp.ones((L,), jnp.int32)
        zeros = jnp.zeros((L,), jnp.int32)

        def per_row(rr, carry0):
            row = wid * RPW + rr
            pltpu.sync_copy(val_hbm.at[row], row_v)

            def build(v, c):
                f = row_v[pl.ds(v * L, L)]
                u = plsc.bitcast(f, jnp.int32)
                key = jnp.where(u < 0, ~u, u ^ jnp.int32(-2147483648))
                key_a[pl.ds(v * L, L)] = key
                idx_a[pl.ds(v * L, L)] = lanes + v * L
                return c

            lax.fori_loop(0, NV, build, 0, unroll=8)

            for p in range(4):
                src_k, src_i = (key_a, idx_a) if p % 2 == 0 else (key_b, idx_b)
                dst_k, dst_i = (key_b, idx_b) if p % 2 == 0 else (key_a, idx_a)
                shift = jnp.int32(8 * p)

                def zero(h, c):
                    hist[pl.ds(h * L, L)] = zeros
                    return c

                lax.fori_loop(0, NBINS, zero, 0, unroll=8)

                def histo(v, c):
                    kv = plsc.load_gather(src_k, [lanes * SEG + v])
                    d = lax.shift_right_logical(kv, shift) & 0xFF
                    plsc.addupdate_scatter(hist, [d * L + lanes], ones)
                    return c

                lax.fori_loop(0, NV, histo, 0, unroll=8)

                def scan_b(h, carry):
                    d = jnp.int32(NBINS - 1) - h
                    hv = plsc.load_gather(hist, [d * L + lanes])
                    incl = plsc.cumsum(hv)
                    excl = incl - hv + carry
                    plsc.store_scatter(off, [d * L + lanes], excl)
                    return carry + jnp.sum(hv)

                lax.fori_loop(0, NBINS, scan_b, jnp.int32(0), unroll=4)

                def perm_pos(v, c):
                    g = lanes * SEG + v
                    kv = plsc.load_gather(src_k, [g])
                    d = lax.shift_right_logical(kv, shift) & 0xFF
                    slot = d * L + lanes
                    pos = plsc.load_gather(off, [slot]) & jnp.int32(N - 1)
                    plsc.store_scatter(pos_buf, [g], pos)
                    plsc.addupdate_scatter(off, [slot], ones)
                    return c

                lax.fori_loop(0, NV, perm_pos, 0, unroll=4)

                def perm_move(v, c):
                    s = pl.ds(v * L, L)
                    pos = pos_buf[s]
                    kv = src_k[s]
                    iv = src_i[s]
                    plsc.store_scatter(dst_k, [pos], kv)
                    plsc.store_scatter(dst_i, [pos], iv)
                    return c

                lax.fori_loop(0, NV, perm_move, 0, unroll=4)

            pltpu.sync_copy(idx_a.at[pl.ds(0, P)], out_hbm.at[row])
            return carry0

        lax.fori_loop(0, RPW, per_row, 0)

    return sortk


def _make_gather(B, D, N, RP):
    info = plsc.get_sparse_core_info()
    NC, NS, L = info.num_cores, info.num_subcores, info.num_lanes
    NW = NC * NS
    d_per_w = D // (NW // B)
    mesh = plsc.VectorSubcoreMesh(core_axis_name="c", subcore_axis_name="s")

    @functools.partial(
        pl.kernel, mesh=mesh,
        out_type=jax.ShapeDtypeStruct((B, D, RP), jnp.float32),
        compiler_params=pltpu.CompilerParams(
            needs_layout_passes=False, use_tc_tiling_on_sc=True),
        scratch_types=[
            pltpu.VMEM((RP,), jnp.int32),
            pltpu.VMEM((N,), jnp.float32),
            pltpu.VMEM((N,), jnp.float32),
            pltpu.VMEM((RP,), jnp.float32),
            pltpu.VMEM((RP,), jnp.float32),
        ],
    )
    def gather(x_hbm, idx_hbm, out_hbm, idx_v, row0_v, row1_v, out0_v, out1_v):
        wid = lax.axis_index("s") * NC + lax.axis_index("c")
        b = wid // (NW // B)
        d0 = (wid % (NW // B)) * d_per_w
        pltpu.sync_copy(idx_hbm.at[b], idx_v)

        def per_d(di, carry):
            d = d0 + di * 2
            pltpu.sync_copy(x_hbm.at[b, d], row0_v)
            pltpu.sync_copy(x_hbm.at[b, d + 1], row1_v)

            def per_j(j, c):
                s = pl.ds(j * L, L)
                idxv = idx_v[s]
                out0_v[s] = plsc.load_gather(row0_v, [idxv])
                out1_v[s] = plsc.load_gather(row1_v, [idxv])
                return c

            lax.fori_loop(0, RP // L, per_j, 0, unroll=16)

            pltpu.sync_copy(out0_v, out_hbm.at[b, d])
            pltpu.sync_copy(out1_v, out_hbm.at[b, d + 1])
            return carry

        lax.fori_loop(0, d_per_w // 2, per_d, 0)

    return gather


def kernel(x, W, b):
    B, D, N = x.shape
    R = W.shape[0]
    P = N // SP_RATIO_K
    val_sort, idx_sort = _project(x, W, b)
    idx_filter = _make_sort(B * R, N, P)(val_sort.reshape(B * R, N))
    feat = _make_gather(B, D, N, R * P)(x, idx_filter.reshape(B, R * P))
    FEAT_star = feat.reshape(B, D, R, P)
    idx_star = jnp.broadcast_to(
        idx_filter.reshape(B, R, P)[:, None, :, :].astype(jnp.float32),
        (B, R, R, P))
    return (FEAT_star, idx_star, idx_sort)


# speedup vs baseline: 738.4129x; 1.0300x over previous
import functools

import jax
import jax.numpy as jnp
from jax import lax
from jax.experimental import pallas as pl
from jax.experimental.pallas import tpu as pltpu
from jax.experimental.pallas import tpu_sc as plsc

SP_RATIO_K = 4


def _proj_body(x_ref, w_ref, b_ref, val_ref, idx_ref):
    x = x_ref[0]
    w = w_ref[...]
    val = lax.dot_general(
        w, x, (((1,), (0,)), ((), ())),
        preferred_element_type=jnp.float32,
    ) + b_ref[...]
    val_ref[0] = val
    r = w.shape[0]
    riota = lax.broadcasted_iota(jnp.int32, val.shape, 0)
    m = jnp.max(val, axis=0, keepdims=True)
    idx = jnp.min(jnp.where(val == m, riota, r), axis=0)
    idx_ref[0, 0] = idx[None, :]


def _project(x, W, b):
    B, D, N = x.shape
    R = W.shape[0]
    NB = 2048
    grid = (B, N // NB)
    val, idx = pl.pallas_call(
        _proj_body,
        grid=grid,
        in_specs=[
            pl.BlockSpec((1, D, NB), lambda i, j: (i, 0, j)),
            pl.BlockSpec((R, D), lambda i, j: (0, 0)),
            pl.BlockSpec((R, 1), lambda i, j: (0, 0)),
        ],
        out_specs=[
            pl.BlockSpec((1, R, NB), lambda i, j: (i, 0, j)),
            pl.BlockSpec((1, 1, 1, NB), lambda i, j: (i, j, 0, 0)),
        ],
        out_shape=[
            jax.ShapeDtypeStruct((B, R, N), jnp.float32),
            jax.ShapeDtypeStruct((B, N // NB, 1, NB), jnp.int32),
        ],
    )(x, W, b.reshape(R, 1))
    return val, idx.reshape(B, N)


def _make_sort(BR, N, P):
    info = plsc.get_sparse_core_info()
    NC, NS, L = info.num_cores, info.num_subcores, info.num_lanes
    NW = NC * NS
    RPW = BR // NW
    SEG = N // L
    NV = N // L
    NBINS = 256
    mesh = plsc.VectorSubcoreMesh(core_axis_name="c", subcore_axis_name="s")

    @functools.partial(
        pl.kernel, mesh=mesh,
        out_type=jax.ShapeDtypeStruct((BR, P), jnp.int32),
        compiler_params=pltpu.CompilerParams(needs_layout_passes=False),
        scratch_types=[
            pltpu.VMEM((N,), jnp.float32),
            pltpu.VMEM((N,), jnp.int32),
            pltpu.VMEM((N,), jnp.int32),
            pltpu.VMEM((N,), jnp.int32),
            pltpu.VMEM((N,), jnp.int32),
            pltpu.VMEM((NBINS * L,), jnp.int32),
            pltpu.VMEM((NBINS * L,), jnp.int32),
            pltpu.VMEM((N,), jnp.int32),
        ],
    )
    def sortk(val_hbm, out_hbm, row_v, key_a, idx_a, key_b, idx_b, hist, off,
              pos_buf):
        wid = lax.axis_index("s") * NC + lax.axis_index("c")
        lanes = lax.iota(jnp.int32, L)
        ones = jnp.ones((L,), jnp.int32)
        zeros = jnp.zeros((L,), jnp.int32)

        def per_row(rr, carry0):
            row = wid * RPW + rr
            pltpu.sync_copy(val_hbm.at[row], row_v)

            def build(v, c):
                f = row_v[pl.ds(v * L, L)]
                u = plsc.bitcast(f, jnp.int32)
                key = jnp.where(u < 0, ~u, u ^ jnp.int32(-2147483648))
                key_a[pl.ds(v * L, L)] = key
                idx_a[pl.ds(v * L, L)] = lanes + v * L
                return c

            lax.fori_loop(0, NV, build, 0, unroll=8)

            for p in range(4):
                src_k, src_i = (key_a, idx_a) if p % 2 == 0 else (key_b, idx_b)
                dst_k, dst_i = (key_b, idx_b) if p % 2 == 0 else (key_a, idx_a)
                shift = jnp.int32(8 * p)

                def zero(h, c):
                    hist[pl.ds(h * L, L)] = zeros
                    return c

                lax.fori_loop(0, NBINS, zero, 0, unroll=8)

                def histo(v, c):
                    kv = plsc.load_gather(src_k, [lanes * SEG + v])
                    d = lax.shift_right_logical(kv, shift) & 0xFF
                    plsc.addupdate_scatter(hist, [d * L + lanes], ones)
                    return c

                lax.fori_loop(0, NV, histo, 0, unroll=8)

                def scan_b(h, carry):
                    d = jnp.int32(NBINS - 1) - h
                    hv = plsc.load_gather(hist, [d * L + lanes])
                    incl = plsc.cumsum(hv)
                    excl = incl - hv + carry
                    plsc.store_scatter(off, [d * L + lanes], excl)
                    return carry + jnp.sum(hv)

                lax.fori_loop(0, NBINS, scan_b, jnp.int32(0), unroll=4)

                def perm_pos(v, c):
                    g = lanes * SEG + v
                    kv = plsc.load_gather(src_k, [g])
                    d = lax.shift_right_logical(kv, shift) & 0xFF
                    slot = d * L + lanes
                    pos = plsc.load_gather(off, [slot]) & jnp.int32(N - 1)
                    plsc.store_scatter(pos_buf, [g], pos)
                    plsc.addupdate_scatter(off, [slot], ones)
                    return c

                lax.fori_loop(0, NV, perm_pos, 0, unroll=4)

                def perm_move(v, c):
                    s = pl.ds(v * L, L)
                    pos = pos_buf[s]
                    kv = src_k[s]
                    iv = src_i[s]
                    plsc.store_scatter(dst_k, [pos], kv)
                    plsc.store_scatter(dst_i, [pos], iv)
                    return c

                lax.fori_loop(0, NV, perm_move, 0, unroll=4)

            pltpu.sync_copy(idx_a.at[pl.ds(0, P)], out_hbm.at[row])
            return carry0

        lax.fori_loop(0, RPW, per_row, 0)

    return sortk


def _idx_star_body(idx_ref, out_ref):
    out_ref[0, 0] = idx_ref[0].astype(jnp.float32)


def _idx_star(idx_filter):
    B, R, P = idx_filter.shape
    return pl.pallas_call(
        _idx_star_body,
        grid=(B, R),
        in_specs=[pl.BlockSpec((1, R, P), lambda i, j: (i, 0, 0))],
        out_specs=pl.BlockSpec((1, 1, R, P), lambda i, j: (i, j, 0, 0)),
        out_shape=jax.ShapeDtypeStruct((B, R, R, P), jnp.float32),
    )(idx_filter)


def _make_gather(B, D, N, RP):
    info = plsc.get_sparse_core_info()
    NC, NS, L = info.num_cores, info.num_subcores, info.num_lanes
    NW = NC * NS
    d_per_w = D // (NW // B)
    mesh = plsc.VectorSubcoreMesh(core_axis_name="c", subcore_axis_name="s")

    @functools.partial(
        pl.kernel, mesh=mesh,
        out_type=jax.ShapeDtypeStruct((B, D, RP), jnp.float32),
        compiler_params=pltpu.CompilerParams(
            needs_layout_passes=False, use_tc_tiling_on_sc=True),
        scratch_types=[
            pltpu.VMEM((RP,), jnp.int32),
            pltpu.VMEM((N,), jnp.float32),
            pltpu.VMEM((N,), jnp.float32),
            pltpu.VMEM((RP,), jnp.float32),
            pltpu.VMEM((RP,), jnp.float32),
            pltpu.SemaphoreType.DMA,
            pltpu.SemaphoreType.DMA,
        ],
    )
    def gather(x_hbm, idx_hbm, out_hbm, idx_v, row0_v, row1_v, out0_v, out1_v,
               sem0, sem1):
        wid = lax.axis_index("s") * NC + lax.axis_index("c")
        b = wid // (NW // B)
        d0 = (wid % (NW // B)) * d_per_w
        pltpu.sync_copy(idx_hbm.at[b], idx_v)

        h0 = h1 = None
        for blk in range(d_per_w // 2):
            d = d0 + blk * 2
            pltpu.sync_copy(x_hbm.at[b, d], row0_v)
            pltpu.sync_copy(x_hbm.at[b, d + 1], row1_v)
            if h0 is not None:
                h0.wait()
                h1.wait()

            def per_j(j, c):
                s = pl.ds(j * L, L)
                idxv = idx_v[s]
                out0_v[s] = plsc.load_gather(row0_v, [idxv])
                out1_v[s] = plsc.load_gather(row1_v, [idxv])
                return c

            lax.fori_loop(0, RP // L, per_j, 0, unroll=16)

            h0 = pltpu.async_copy(out0_v, out_hbm.at[b, d], sem0)
            h1 = pltpu.async_copy(out1_v, out_hbm.at[b, d + 1], sem1)
        h0.wait()
        h1.wait()

    return gather


def kernel(x, W, b):
    B, D, N = x.shape
    R = W.shape[0]
    P = N // SP_RATIO_K
    val_sort, idx_sort = _project(x, W, b)
    idx_filter = _make_sort(B * R, N, P)(val_sort.reshape(B * R, N))
    feat = _make_gather(B, D, N, R * P)(x, idx_filter.reshape(B, R * P))
    FEAT_star = feat.reshape(B, D, R, P)
    idx_star = _idx_star(idx_filter.reshape(B, R, P))
    return (FEAT_star, idx_star, idx_sort)
